# baseline (speedup 1.0000x reference)
import jax
import jax.numpy as jnp
from jax.experimental import pallas as pl
from jax.experimental.pallas import tpu as pltpu

_N = 4
_C = 19
_H = 512
_W = 512
_G = 128
_SH = 8
_NG = _H // _G
_RATIO = 0.2
_TOTPOW = float(_H * _W) ** (1.0 - _RATIO)


def _fold(x):
    return x[:, 0:128] + x[:, 128:256] + x[:, 256:384] + x[:, 384:512]


def _acc_kernel(prob_ref, loss_ref, p_scr, lab_scr, acc_ref):
    j = pl.program_id(0)
    i = pl.program_id(1)
    jm = j % 2
    jp = (j + 1) % 2

    @pl.when((j == 0) & (i == 0))
    def _init_acc():
        acc_ref[...] = jnp.zeros_like(acc_ref)

    @pl.when((j < _NG) & (i == 0))
    def _init_p():
        p_scr[jm] = jnp.zeros_like(p_scr[jm])

    @pl.when(j < _NG)
    def _channel():
        for s in range(_G // _SH):
            r0 = s * _SH
            v0 = prob_ref[0, 0, r0 : r0 + _SH]
            maxv = v0
            arg = jnp.zeros((_SH, _W), jnp.int32)
            q = [v0 * v0, None, None, None]
            for c in range(1, _C):
                v = prob_ref[0, c, r0 : r0 + _SH]
                gt = v > maxv
                maxv = jnp.where(gt, v, maxv)
                arg = jnp.where(gt, jnp.int32(c), arg)
                r = c % 4
                sq = v * v
                q[r] = sq if q[r] is None else q[r] + sq
            lab_scr[jm, i, r0 : r0 + _SH] = arg
            for r in range(4):
                m = (r - i) % 4
                p_scr[jm, m, r0 : r0 + _SH] += q[r]

    @pl.when(j >= 1)
    def _classify():
        zero = jnp.zeros((_SH, _W), jnp.float32)
        one = jnp.ones((_SH, _W), jnp.float32)
        for s in range(_G // _SH):
            r0 = s * _SH
            lab = lab_scr[jp, i, r0 : r0 + _SH]
            pm = p_scr[jp, i, r0 : r0 + _SH]
            for c in range(_C):
                msk = lab == c
                acc_ref[i * _C + c] += _fold(jnp.where(msk, pm, zero))
                acc_ref[_N * _C + i * _C + c] += _fold(jnp.where(msk, one, zero))

    @pl.when((j == _NG) & (i == _N - 1))
    def _finalize():
        s = jnp.sum(acc_ref[...], axis=(1, 2))
        a = s[: _N * _C]
        cnt = s[_N * _C :]
        wv = 1.0 / jnp.maximum(cnt ** _RATIO * _TOTPOW, 1.0)
        num = jnp.sum(a * wv)
        wsum = jnp.float32(_C) * jnp.sum(cnt * wv)
        loss_ref[0, 0] = -num / (_N * _C * wsum)


@jax.jit
def kernel(pred, prob):
    del pred
    loss = pl.pallas_call(
        _acc_kernel,
        grid=(_NG + 1, _N),
        in_specs=[
            pl.BlockSpec(
                (1, _C, _G, _W),
                lambda j, i: (i, 0, jnp.minimum(j, _NG - 1), 0),
            ),
        ],
        out_specs=pl.BlockSpec(memory_space=pltpu.SMEM),
        out_shape=jax.ShapeDtypeStruct((1, 1), jnp.float32),
        scratch_shapes=[
            pltpu.VMEM((2, _N, _G, _W), jnp.float32),
            pltpu.VMEM((2, _N, _G, _W), jnp.int32),
            pltpu.VMEM((2 * _N * _C, _SH, 128), jnp.float32),
        ],
    )(prob)
    return loss[0, 0]
